# baseline (speedup 1.0000x reference)
import functools

import jax
import jax.numpy as jnp
from jax import lax
from jax.experimental import pallas as pl
from jax.experimental.pallas import tpu as pltpu
from jax.experimental.pallas import tpu_sc as plsc

D = 1024
T = 64
TPC = 8
C = T // TPC
G = 16
DS = max(D // T, 4)

BT = 1024

NC = 2
NS = 16
NW = NC * NS


def _tc_body(x_ref, w_ref, b_ref, s16_ref,
             h_ref, ti_ref, rw_ref, mn_ref, mx_ref,
             wsig_ref, wclu_ref):
    i = pl.program_id(0)

    @pl.when(i == 0)
    def _():
        q = jnp.broadcast_to(s16_ref[...][None], (T, DS, T)).reshape(D, T)
        dcol = lax.broadcasted_iota(jnp.int32, (D, T), 0) // DS
        trow = lax.broadcasted_iota(jnp.int32, (D, T), 1)
        wfull = jnp.where(dcol == trow, q, 0.0)
        wsig_ref[...] = wfull.astype(jnp.bfloat16)
        qd = jnp.sum(wfull, axis=1, keepdims=True)
        ccol = lax.broadcasted_iota(jnp.int32, (D, C), 1)
        drow = lax.broadcasted_iota(jnp.int32, (D, C), 0) // (D // C)
        wclu_ref[...] = jnp.where(drow == ccol, qd, 0.0).astype(jnp.bfloat16)

    x = x_ref[...]
    mu = jnp.sum(x, axis=1, keepdims=True) * (1.0 / D)
    xc = x - mu
    var = jnp.sum(xc * xc, axis=1, keepdims=True) * (1.0 / D)
    h = xc * lax.rsqrt(var + 1e-5) * w_ref[...] + b_ref[...]
    for c in range(D // 128):
        h_ref[:, c] = h[:, c * 128:(c + 1) * 128].reshape(BT // 8, 8, 128)

    shb = ((h > 0.0).astype(jnp.bfloat16) - (h < 0.0).astype(jnp.bfloat16))
    scores = jnp.dot(shb, wsig_ref[...], preferred_element_type=jnp.float32)
    cs = jnp.dot(shb, wclu_ref[...], preferred_element_type=jnp.float32)

    ci = lax.broadcasted_iota(jnp.int32, (BT, C), 1)
    m1 = jnp.max(cs, axis=1, keepdims=True)
    bc = jnp.min(jnp.where(cs == m1, ci, C), axis=1, keepdims=True)

    ti_iota = lax.broadcasted_iota(jnp.int32, (BT, T), 1)
    keep = (ti_iota // TPC) == bc
    masked = jnp.where(keep, scores, -1e9)
    m2 = jnp.max(masked, axis=1, keepdims=True)
    ti = jnp.min(jnp.where(masked == m2, ti_iota, T), axis=1, keepdims=True)

    sumexp = jnp.sum(jnp.exp(masked - m2), axis=1, keepdims=True)
    rw = 1.0 / sumexp

    ti_ref[...] = ti.reshape(BT // 128, 128)
    rw_ref[...] = rw.reshape(BT // 128, 128)

    dcol = lax.broadcasted_iota(jnp.int32, (BT, D), 1) // DS
    own = dcol == ti
    rowmin = jnp.min(jnp.where(own, h, 1e30), axis=1, keepdims=True)
    rowmax = jnp.max(jnp.where(own, h, -1e30), axis=1, keepdims=True)

    oh = ti_iota == ti
    mnp = jnp.min(jnp.where(oh, rowmin, 1e30), axis=0, keepdims=True)
    mxp = jnp.max(jnp.where(oh, rowmax, -1e30), axis=0, keepdims=True)

    @pl.when(i == 0)
    def _():
        mn_ref[...] = jnp.full((1, T), 1e30, jnp.float32)
        mx_ref[...] = jnp.full((1, T), -1e30, jnp.float32)

    mn_ref[...] = jnp.minimum(mn_ref[...], mnp)
    mx_ref[...] = jnp.maximum(mx_ref[...], mxp)


def _tc_route(x2d, w, b, s16):
    N = x2d.shape[0]
    nb = N // BT
    return pl.pallas_call(
        _tc_body,
        grid=(nb,),
        in_specs=[
            pl.BlockSpec((BT, D), lambda i: (i, 0)),
            pl.BlockSpec((1, D), lambda i: (0, 0)),
            pl.BlockSpec((1, D), lambda i: (0, 0)),
            pl.BlockSpec((DS, T), lambda i: (0, 0)),
        ],
        scratch_shapes=[
            pltpu.VMEM((D, T), jnp.bfloat16),
            pltpu.VMEM((D, C), jnp.bfloat16),
        ],
        out_specs=[
            pl.BlockSpec((BT // 8, 8, 8, 128), lambda i: (i, 0, 0, 0)),
            pl.BlockSpec((BT // 128, 128), lambda i: (i, 0)),
            pl.BlockSpec((BT // 128, 128), lambda i: (i, 0)),
            pl.BlockSpec((1, T), lambda i: (0, 0)),
            pl.BlockSpec((1, T), lambda i: (0, 0)),
        ],
        out_shape=[
            jax.ShapeDtypeStruct((N // 8, 8, 8, 128), jnp.float32),
            jax.ShapeDtypeStruct((N // 128, 128), jnp.int32),
            jax.ShapeDtypeStruct((N // 128, 128), jnp.float32),
            jax.ShapeDtypeStruct((1, T), jnp.float32),
            jax.ShapeDtypeStruct((1, T), jnp.float32),
        ],
    )(x2d, w.reshape(1, D), b.reshape(1, D), s16)


def _sc_body(TPW, out_ref, ti_ref, rw_ref, mn_ref, mx_ref,
             bas_ref, slo_ref,
             idx0, idx1, rows0, rows1, ti_v, rw_v, mn_v, mx_v,
             bas_v, slo_v, sem):
    wid = lax.axis_index("s") * NC + lax.axis_index("c")
    base = wid * TPW

    pltpu.sync_copy(ti_ref.at[pl.ds(base, TPW)], ti_v)
    pltpu.sync_copy(rw_ref.at[pl.ds(base, TPW)], rw_v)
    pltpu.sync_copy(mn_ref, mn_v)
    pltpu.sync_copy(mx_ref, mx_v)
    pltpu.sync_copy(bas_ref, bas_v)
    pltpu.sync_copy(slo_ref, slo_v)

    lane = lax.iota(jnp.int32, 16)
    ts_per_row = 128 // DS

    for cch, idx_v in enumerate((idx0, idx1)):
        for j in range(8):
            tl = cch * 128 + j * 16
            nv = base + tl + lane
            tv = ti_v[pl.ds(tl, 16)]
            ridx = ((nv // 8) * (8 * ts_per_row)
                    + (tv // ts_per_row) * 8 + nv % 8)
            idx_v[pl.ds(j * 16, 16)] = ridx

    cp0 = pltpu.async_copy(out_ref.at[idx0], rows0, sem)
    cp1 = pltpu.async_copy(out_ref.at[idx1], rows1, sem)
    cp0.wait()
    cp1.wait()

    for cch, rows_v in enumerate((rows0, rows1)):

        @plsc.parallel_loop(0, 128, unroll=4)
        def _(tr):
            tl = cch * 128 + tr
            tlb = jnp.full((DS,), tl, jnp.int32)
            tvb = plsc.load_gather(ti_v, [tlb])
            rwb = plsc.load_gather(rw_v, [tlb])
            mnb = plsc.load_gather(mn_v, [tvb])
            mxb = plsc.load_gather(mx_v, [tvb])
            t0 = tvb[0]
            off = (t0 % ts_per_row) * DS
            den = mxb - mnb + 1e-8
            xs = rows_v[tr, pl.ds(off, DS)]
            xn = jnp.clip((xs - mnb) / den, 0.0, 1.0 - 1e-6)
            gi = jnp.clip((xn * G).astype(jnp.int32), 0, G - 1)
            fidx = tvb * (DS * G) + lane * G + gi
            bb = plsc.load_gather(bas_v, [fidx])
            ss = plsc.load_gather(slo_v, [fidx])
            xl = xn * G - gi.astype(jnp.float32)
            rows_v[tr, pl.ds(off, DS)] = xs + (bb + ss * xl) * rwb

    cp0 = pltpu.async_copy(rows0, out_ref.at[idx0], sem)
    cp1 = pltpu.async_copy(rows1, out_ref.at[idx1], sem)
    cp0.wait()
    cp1.wait()


def _sc_patch(out_rows_ref, ti, rw, mn, mx, basf, slof):
    N = rw.size
    TPW = N // NW
    mesh = plsc.VectorSubcoreMesh(
        core_axis_name="c", subcore_axis_name="s",
        num_cores=NC, num_subcores=NS)
    fn = pl.kernel(
        functools.partial(_sc_body, TPW),
        out_type=(),
        mesh=mesh,
        compiler_params=pltpu.CompilerParams(needs_layout_passes=False),
        scratch_types=[
            pltpu.VMEM((128,), jnp.int32),
            pltpu.VMEM((128,), jnp.int32),
            pltpu.VMEM((128, 128), jnp.float32),
            pltpu.VMEM((128, 128), jnp.float32),
            pltpu.VMEM((TPW,), jnp.int32),
            pltpu.VMEM((TPW,), jnp.float32),
            pltpu.VMEM((T,), jnp.float32),
            pltpu.VMEM((T,), jnp.float32),
            pltpu.VMEM((T * DS * G,), jnp.float32),
            pltpu.VMEM((T * DS * G,), jnp.float32),
            pltpu.SemaphoreType.DMA,
        ],
    )
    fn(out_rows_ref, ti.reshape(N), rw.reshape(N),
       mn.reshape(T), mx.reshape(T), basf, slof)


def kernel(x, ln_weight, ln_bias, spline_bases, spline_slopes, output_scale):
    B, S, Dm = x.shape
    N = B * S

    s16 = jnp.sign(jnp.mean(spline_slopes, axis=-1)).T

    x2d = x.reshape(N, Dm)
    h, ti, rw, mn, mx = _tc_route(x2d, ln_weight, ln_bias, s16)

    osc = output_scale[:, None, None]
    basf = (spline_bases * osc).reshape(-1)
    slof = (spline_slopes * osc).reshape(-1)

    out_rows = jax.new_ref(h.reshape(N * (Dm // 128), 128))
    _sc_patch(out_rows, ti, rw, mn, mx, basf, slof)
    out = out_rows[...].reshape(N // 8, 8, Dm // 128, 128)
    return out.transpose(0, 2, 1, 3).reshape(B, S, Dm)


# speedup vs baseline: 1.0124x; 1.0124x over previous
import functools

import jax
import jax.numpy as jnp
from jax import lax
from jax.experimental import pallas as pl
from jax.experimental.pallas import tpu as pltpu
from jax.experimental.pallas import tpu_sc as plsc

D = 1024
T = 64
TPC = 8
C = T // TPC
G = 16
DS = max(D // T, 4)

BT = 2048

NC = 2
NS = 16
NW = NC * NS


def _tc_body(x_ref, w_ref, b_ref, s16_ref,
             h_ref, ti_ref, rw_ref, mn_ref, mx_ref,
             wsig_ref, wclu_ref):
    i = pl.program_id(0)

    @pl.when(i == 0)
    def _():
        q = jnp.broadcast_to(s16_ref[...][None], (T, DS, T)).reshape(D, T)
        dcol = lax.broadcasted_iota(jnp.int32, (D, T), 0) // DS
        trow = lax.broadcasted_iota(jnp.int32, (D, T), 1)
        wfull = jnp.where(dcol == trow, q, 0.0)
        wsig_ref[...] = wfull.astype(jnp.bfloat16)
        qd = jnp.sum(wfull, axis=1, keepdims=True)
        ccol = lax.broadcasted_iota(jnp.int32, (D, C), 1)
        drow = lax.broadcasted_iota(jnp.int32, (D, C), 0) // (D // C)
        wclu_ref[...] = jnp.where(drow == ccol, qd, 0.0).astype(jnp.bfloat16)

    x = x_ref[...]
    mu = jnp.sum(x, axis=1, keepdims=True) * (1.0 / D)
    xc = x - mu
    var = jnp.sum(xc * xc, axis=1, keepdims=True) * (1.0 / D)
    h = xc * lax.rsqrt(var + 1e-5) * w_ref[...] + b_ref[...]
    for c in range(D // 128):
        h_ref[:, c] = h[:, c * 128:(c + 1) * 128].reshape(BT // 8, 8, 128)

    shb = ((h > 0.0).astype(jnp.bfloat16) - (h < 0.0).astype(jnp.bfloat16))
    scores = jnp.dot(shb, wsig_ref[...], preferred_element_type=jnp.float32)
    cs = jnp.dot(shb, wclu_ref[...], preferred_element_type=jnp.float32)

    ci = lax.broadcasted_iota(jnp.int32, (BT, C), 1)
    m1 = jnp.max(cs, axis=1, keepdims=True)
    bc = jnp.min(jnp.where(cs == m1, ci, C), axis=1, keepdims=True)

    ti_iota = lax.broadcasted_iota(jnp.int32, (BT, T), 1)
    keep = (ti_iota // TPC) == bc
    masked = jnp.where(keep, scores, -1e9)
    m2 = jnp.max(masked, axis=1, keepdims=True)
    ti = jnp.min(jnp.where(masked == m2, ti_iota, T), axis=1, keepdims=True)

    sumexp = jnp.sum(jnp.exp(masked - m2), axis=1, keepdims=True)
    rw = 1.0 / sumexp

    ti_ref[...] = ti.reshape(BT // 128, 128)
    rw_ref[...] = rw.reshape(BT // 128, 128)

    dcol = lax.broadcasted_iota(jnp.int32, (BT, D), 1) // DS
    own = dcol == ti
    rowmin = jnp.min(jnp.where(own, h, 1e30), axis=1, keepdims=True)
    rowmax = jnp.max(jnp.where(own, h, -1e30), axis=1, keepdims=True)

    oh = ti_iota == ti
    mnp = jnp.min(jnp.where(oh, rowmin, 1e30), axis=0, keepdims=True)
    mxp = jnp.max(jnp.where(oh, rowmax, -1e30), axis=0, keepdims=True)

    @pl.when(i == 0)
    def _():
        mn_ref[...] = jnp.full((1, T), 1e30, jnp.float32)
        mx_ref[...] = jnp.full((1, T), -1e30, jnp.float32)

    mn_ref[...] = jnp.minimum(mn_ref[...], mnp)
    mx_ref[...] = jnp.maximum(mx_ref[...], mxp)


def _tc_route(x2d, w, b, s16):
    N = x2d.shape[0]
    nb = N // BT
    return pl.pallas_call(
        _tc_body,
        grid=(nb,),
        in_specs=[
            pl.BlockSpec((BT, D), lambda i: (i, 0)),
            pl.BlockSpec((1, D), lambda i: (0, 0)),
            pl.BlockSpec((1, D), lambda i: (0, 0)),
            pl.BlockSpec((DS, T), lambda i: (0, 0)),
        ],
        scratch_shapes=[
            pltpu.VMEM((D, T), jnp.bfloat16),
            pltpu.VMEM((D, C), jnp.bfloat16),
        ],
        out_specs=[
            pl.BlockSpec((BT // 8, 8, 8, 128), lambda i: (i, 0, 0, 0)),
            pl.BlockSpec((BT // 128, 128), lambda i: (i, 0)),
            pl.BlockSpec((BT // 128, 128), lambda i: (i, 0)),
            pl.BlockSpec((1, T), lambda i: (0, 0)),
            pl.BlockSpec((1, T), lambda i: (0, 0)),
        ],
        out_shape=[
            jax.ShapeDtypeStruct((N // 8, 8, 8, 128), jnp.float32),
            jax.ShapeDtypeStruct((N // 128, 128), jnp.int32),
            jax.ShapeDtypeStruct((N // 128, 128), jnp.float32),
            jax.ShapeDtypeStruct((1, T), jnp.float32),
            jax.ShapeDtypeStruct((1, T), jnp.float32),
        ],
    )(x2d, w.reshape(1, D), b.reshape(1, D), s16)


def _sc_body(TPW, out_ref, ti_ref, rw_ref, mn_ref, mx_ref,
             bas_ref, slo_ref,
             idx0, idx1, rows0, rows1, ti_v, rw_v, mn_v, mx_v,
             bas_v, slo_v, sem):
    wid = lax.axis_index("s") * NC + lax.axis_index("c")
    base = wid * TPW

    stage = [
        pltpu.async_copy(ti_ref.at[pl.ds(base, TPW)], ti_v, sem),
        pltpu.async_copy(rw_ref.at[pl.ds(base, TPW)], rw_v, sem),
        pltpu.async_copy(mn_ref, mn_v, sem),
        pltpu.async_copy(mx_ref, mx_v, sem),
        pltpu.async_copy(bas_ref, bas_v, sem),
        pltpu.async_copy(slo_ref, slo_v, sem),
    ]
    for cp in stage:
        cp.wait()

    lane = lax.iota(jnp.int32, 16)
    ts_per_row = 128 // DS

    for cch, idx_v in enumerate((idx0, idx1)):
        for j in range(8):
            tl = cch * 128 + j * 16
            nv = base + tl + lane
            tv = ti_v[pl.ds(tl, 16)]
            ridx = ((nv // 8) * (8 * ts_per_row)
                    + (tv // ts_per_row) * 8 + nv % 8)
            idx_v[pl.ds(j * 16, 16)] = ridx

    cp0 = pltpu.async_copy(out_ref.at[idx0], rows0, sem)
    cp1 = pltpu.async_copy(out_ref.at[idx1], rows1, sem)
    cp0.wait()
    cp1.wait()

    for cch, rows_v in enumerate((rows0, rows1)):

        @plsc.parallel_loop(0, 128, unroll=4)
        def _(tr):
            tl = cch * 128 + tr
            tlb = jnp.full((DS,), tl, jnp.int32)
            tvb = plsc.load_gather(ti_v, [tlb])
            rwb = plsc.load_gather(rw_v, [tlb])
            mnb = plsc.load_gather(mn_v, [tvb])
            mxb = plsc.load_gather(mx_v, [tvb])
            t0 = tvb[0]
            off = (t0 % ts_per_row) * DS
            den = mxb - mnb + 1e-8
            xs = rows_v[tr, pl.ds(off, DS)]
            xn = jnp.clip((xs - mnb) / den, 0.0, 1.0 - 1e-6)
            gi = jnp.clip((xn * G).astype(jnp.int32), 0, G - 1)
            fidx = tvb * (DS * G) + lane * G + gi
            bb = plsc.load_gather(bas_v, [fidx])
            ss = plsc.load_gather(slo_v, [fidx])
            xl = xn * G - gi.astype(jnp.float32)
            rows_v[tr, pl.ds(off, DS)] = xs + (bb + ss * xl) * rwb

    cp0 = pltpu.async_copy(rows0, out_ref.at[idx0], sem)
    cp1 = pltpu.async_copy(rows1, out_ref.at[idx1], sem)
    cp0.wait()
    cp1.wait()


def _sc_patch(out_rows_ref, ti, rw, mn, mx, basf, slof):
    N = rw.size
    TPW = N // NW
    mesh = plsc.VectorSubcoreMesh(
        core_axis_name="c", subcore_axis_name="s",
        num_cores=NC, num_subcores=NS)
    fn = pl.kernel(
        functools.partial(_sc_body, TPW),
        out_type=(),
        mesh=mesh,
        compiler_params=pltpu.CompilerParams(needs_layout_passes=False),
        scratch_types=[
            pltpu.VMEM((128,), jnp.int32),
            pltpu.VMEM((128,), jnp.int32),
            pltpu.VMEM((128, 128), jnp.float32),
            pltpu.VMEM((128, 128), jnp.float32),
            pltpu.VMEM((TPW,), jnp.int32),
            pltpu.VMEM((TPW,), jnp.float32),
            pltpu.VMEM((T,), jnp.float32),
            pltpu.VMEM((T,), jnp.float32),
            pltpu.VMEM((T * DS * G,), jnp.float32),
            pltpu.VMEM((T * DS * G,), jnp.float32),
            pltpu.SemaphoreType.DMA,
        ],
    )
    fn(out_rows_ref, ti.reshape(N), rw.reshape(N),
       mn.reshape(T), mx.reshape(T), basf, slof)


def kernel(x, ln_weight, ln_bias, spline_bases, spline_slopes, output_scale):
    B, S, Dm = x.shape
    N = B * S

    s16 = jnp.sign(jnp.mean(spline_slopes, axis=-1)).T

    x2d = x.reshape(N, Dm)
    h, ti, rw, mn, mx = _tc_route(x2d, ln_weight, ln_bias, s16)

    osc = output_scale[:, None, None]
    basf = (spline_bases * osc).reshape(-1)
    slof = (spline_slopes * osc).reshape(-1)

    out_rows = jax.new_ref(h.reshape(N * (Dm // 128), 128))
    _sc_patch(out_rows, ti, rw, mn, mx, basf, slof)
    out = out_rows[...].reshape(N // 8, 8, Dm // 128, 128)
    return out.transpose(0, 2, 1, 3).reshape(B, S, Dm)
